# Initial kernel scaffold; baseline (speedup 1.0000x reference)
import jax
import jax.numpy as jnp
from jax.experimental import pallas as pl


def kernel(probs, gumbel_u, mask_len):
    raise NotImplementedError("write your pallas kernel here")


# speedup vs baseline: 57.3163x; 57.3163x over previous
import jax
import jax.numpy as jnp
from jax import lax
from jax.experimental import pallas as pl
from jax.experimental.pallas import tpu as pltpu

_ROWS_PER_BLOCK = 8
_N = 32768


def _mask_kernel(k_ref, probs_ref, gumbel_ref, out_ref):
    k = k_ref[0]
    p = probs_ref[...]
    u = gumbel_ref[...]

    eps = 1e-20
    inner = -jnp.log(jnp.maximum(u, eps))
    gumbel_noise = -jnp.log(jnp.maximum(inner, eps))
    conf = jnp.log(p + 1e-05) + gumbel_noise

    bits = lax.bitcast_convert_type(conf, jnp.uint32)
    flip = jnp.where(
        (bits >> 31) == jnp.uint32(1),
        jnp.uint32(0xFFFFFFFF),
        jnp.uint32(0x80000000),
    )
    ukey = bits ^ flip

    rows = p.shape[0]

    def bit_body(_, carry):
        tpref, bit = carry
        cand = tpref | bit
        cnt = jnp.sum((ukey < cand).astype(jnp.int32), axis=1, keepdims=True)
        tpref = jnp.where(cnt >= k, tpref, cand)
        return tpref, bit >> jnp.uint32(1)

    t0 = jnp.zeros((rows, 1), jnp.uint32)
    T, _ = lax.fori_loop(0, 32, bit_body, (t0, jnp.uint32(0x80000000)))

    lt = ukey < T
    c_lt = jnp.sum(lt.astype(jnp.int32), axis=1, keepdims=True)
    need = k - c_lt
    eq = ukey == T
    idx = lax.broadcasted_iota(jnp.int32, (rows, _N), 1)

    def tie_body(_, carry):
        pref, bit = carry
        cand = pref | bit
        cnt = jnp.sum((eq & (idx < cand)).astype(jnp.int32), axis=1, keepdims=True)
        pref = jnp.where(cnt >= need, pref, cand)
        return pref, bit >> 1

    p0 = jnp.zeros((rows, 1), jnp.int32)
    P, _ = lax.fori_loop(0, 15, tie_body, (p0, jnp.int32(1 << 14)))

    out_ref[...] = lt | (eq & (idx <= P) & (need > 0))


def kernel(probs, gumbel_u, mask_len):
    rows, n = probs.shape
    assert n == _N
    k = jnp.asarray(mask_len, jnp.int32).reshape(1)
    grid = (rows // _ROWS_PER_BLOCK,)
    out = pl.pallas_call(
        _mask_kernel,
        grid=grid,
        in_specs=[
            pl.BlockSpec(memory_space=pltpu.SMEM),
            pl.BlockSpec((_ROWS_PER_BLOCK, _N), lambda i: (i, 0)),
            pl.BlockSpec((_ROWS_PER_BLOCK, _N), lambda i: (i, 0)),
        ],
        out_specs=pl.BlockSpec((_ROWS_PER_BLOCK, _N), lambda i: (i, 0)),
        out_shape=jax.ShapeDtypeStruct((rows, n), jnp.bool_),
    )(k, probs, gumbel_u)
    return out


# speedup vs baseline: 78.2528x; 1.3653x over previous
import jax
import jax.numpy as jnp
from jax import lax
from jax.experimental import pallas as pl
from jax.experimental.pallas import tpu as pltpu

_ROWS_PER_BLOCK = 32
_N = 32768
_LANES = 128
_CHUNKS = _N // _LANES


def _mask_kernel(k_ref, probs_ref, gumbel_ref, out_ref):
    k = k_ref[0]
    p = probs_ref[...]
    u = gumbel_ref[...]

    eps = 1e-20
    inner = -jnp.log(jnp.maximum(u, eps))
    gumbel_noise = -jnp.log(jnp.maximum(inner, eps))
    conf = jnp.log(p + 1e-05) + gumbel_noise

    bits = lax.bitcast_convert_type(conf, jnp.uint32)
    flip = jnp.where(
        (bits >> 31) == jnp.uint32(1),
        jnp.uint32(0xFFFFFFFF),
        jnp.uint32(0x80000000),
    )
    ukey = bits ^ flip

    rows = p.shape[0]
    ones_col = jnp.ones((_N, 1), jnp.float32)
    k_f = k.astype(jnp.float32)

    def bit_body(_, carry):
        tpref, bit = carry
        cand = tpref | bit
        ltf = jnp.where(ukey < cand, 1.0, 0.0)
        cnt = jax.lax.dot_general(
            ltf, ones_col, (((1,), (0,)), ((), ())),
            preferred_element_type=jnp.float32,
        )
        tpref = jnp.where(cnt >= k_f, tpref, cand)
        return tpref, bit >> jnp.uint32(1)

    t0 = jnp.zeros((rows, 1), jnp.uint32)
    T, _ = lax.fori_loop(0, 32, bit_body, (t0, jnp.uint32(0x80000000)))

    lt = ukey < T
    ltf = jnp.where(lt, 1.0, 0.0)
    c_lt = jax.lax.dot_general(
        ltf, ones_col, (((1,), (0,)), ((), ())),
        preferred_element_type=jnp.float32,
    )
    need = k_f - c_lt
    eq = ukey == T

    eqf = jnp.where(eq, 1.0, 0.0)
    e2 = eqf.reshape(rows * _CHUNKS, _LANES)
    li = lax.broadcasted_iota(jnp.int32, (_LANES, _LANES), 0)
    lj = lax.broadcasted_iota(jnp.int32, (_LANES, _LANES), 1)
    lt_incl = jnp.where(li <= lj, 1.0, 0.0)
    intra = jax.lax.dot_general(
        e2, lt_incl, (((1,), (0,)), ((), ())),
        preferred_element_type=jnp.float32,
    )
    totals = jax.lax.dot_general(
        e2, jnp.ones((_LANES, 1), jnp.float32), (((1,), (0,)), ((), ())),
        preferred_element_type=jnp.float32,
    ).reshape(rows, _CHUNKS)
    ci = lax.broadcasted_iota(jnp.int32, (_CHUNKS, _CHUNKS), 0)
    cj = lax.broadcasted_iota(jnp.int32, (_CHUNKS, _CHUNKS), 1)
    slt = jnp.where(ci < cj, 1.0, 0.0)
    offs = jax.lax.dot_general(
        totals, slt, (((1,), (0,)), ((), ())),
        preferred_element_type=jnp.float32,
    )
    rank = intra.reshape(rows, _CHUNKS, _LANES) + offs[:, :, None]
    rank = rank.reshape(rows, _N)

    out_ref[...] = lt | (eq & (rank <= need))


def kernel(probs, gumbel_u, mask_len):
    rows, n = probs.shape
    assert n == _N
    k = jnp.asarray(mask_len, jnp.int32).reshape(1)
    grid = (rows // _ROWS_PER_BLOCK,)
    out = pl.pallas_call(
        _mask_kernel,
        grid=grid,
        in_specs=[
            pl.BlockSpec(memory_space=pltpu.SMEM),
            pl.BlockSpec((_ROWS_PER_BLOCK, _N), lambda i: (i, 0)),
            pl.BlockSpec((_ROWS_PER_BLOCK, _N), lambda i: (i, 0)),
        ],
        out_specs=pl.BlockSpec((_ROWS_PER_BLOCK, _N), lambda i: (i, 0)),
        out_shape=jax.ShapeDtypeStruct((rows, n), jnp.bool_),
    )(k, probs, gumbel_u)
    return out
